# baseline (device time: 336976 ns/iter reference)
import jax
import jax.numpy as jnp
from jax import lax
from jax.experimental import pallas as pl
from jax.experimental.pallas import tpu as pltpu


def kernel(Q, K, V):
    b, s, h, d = Q.shape
    scale = d ** -0.5

    def body(q_ref, k_ref, v_ref, k_any, v_any, out_ref, krem, vrem,
             ksx, krx, ksy, kry, vsx, vrx, vsy, vry):
        hh = pl.program_id(0)
        my_x = lax.axis_index("x")
        my_y = lax.axis_index("y")
        my_z = lax.axis_index("z")
        xnbr = (1 - my_x, my_y, my_z)
        ynbr = (my_x, 1 - my_y, my_z)

        def x_descriptors(i):
            k_copy = pltpu.make_async_remote_copy(
                src_ref=k_any.at[0, :, i, :],
                dst_ref=krem.at[i],
                send_sem=ksx.at[i // 2],
                recv_sem=krx.at[i // 2],
                device_id=xnbr,
                device_id_type=pl.DeviceIdType.MESH,
            )
            v_copy = pltpu.make_async_remote_copy(
                src_ref=v_any.at[0, :, i, :],
                dst_ref=vrem.at[i],
                send_sem=vsx.at[i // 2],
                recv_sem=vrx.at[i // 2],
                device_id=xnbr,
                device_id_type=pl.DeviceIdType.MESH,
            )
            return k_copy, v_copy

        def y_descriptors(i):
            k_copy = pltpu.make_async_remote_copy(
                src_ref=krem.at[i],
                dst_ref=krem.at[i],
                send_sem=ksy.at[i // 2],
                recv_sem=kry.at[i // 2],
                device_id=ynbr,
                device_id_type=pl.DeviceIdType.MESH,
            )
            v_copy = pltpu.make_async_remote_copy(
                src_ref=vrem.at[i],
                dst_ref=vrem.at[i],
                send_sem=vsy.at[i // 2],
                recv_sem=vry.at[i // 2],
                device_id=ynbr,
                device_id_type=pl.DeviceIdType.MESH,
            )
            return k_copy, v_copy

        @pl.when(hh == 0)
        def _():
            barrier = pltpu.get_barrier_semaphore()
            for nbr in (xnbr, ynbr):
                pl.semaphore_signal(barrier, inc=1, device_id=nbr,
                                    device_id_type=pl.DeviceIdType.MESH)
            pl.semaphore_wait(barrier, 2)
            for i in range(h):
                @pl.when((i % 2) == my_y)
                def _():
                    kc, vc = x_descriptors(i)
                    kc.start()
                    vc.start()

        q = q_ref[0, :, hh, :]
        k1 = k_ref[0, :, hh, :]
        v1 = v_ref[0, :, hh, :]
        s1 = lax.dot_general(q, k1, (((1,), (1,)), ((), ())),
                             preferred_element_type=jnp.float32) * scale
        m1 = jnp.max(s1, axis=1, keepdims=True)
        p1 = jnp.exp(s1 - m1)
        l1 = jnp.sum(p1, axis=1, keepdims=True)
        o1 = lax.dot_general(p1, v1, (((1,), (0,)), ((), ())),
                             preferred_element_type=jnp.float32)

        @pl.when((hh % 2) == my_y)
        def _():
            kc, vc = x_descriptors(hh)
            kc.wait_recv()
            vc.wait_recv()
            rk, rv = y_descriptors(hh)
            rk.start()
            rv.start()

        @pl.when((hh % 2) != my_y)
        def _():
            kc, vc = y_descriptors(hh)
            kc.wait_recv()
            vc.wait_recv()

        k2 = krem[hh]
        v2 = vrem[hh]
        s2 = lax.dot_general(q, k2, (((1,), (1,)), ((), ())),
                             preferred_element_type=jnp.float32) * scale
        m2 = jnp.max(s2, axis=1, keepdims=True)
        p2 = jnp.exp(s2 - m2)
        l2 = jnp.sum(p2, axis=1, keepdims=True)
        o2 = lax.dot_general(p2, v2, (((1,), (0,)), ((), ())),
                             preferred_element_type=jnp.float32)

        m = jnp.maximum(m1, m2)
        a1 = jnp.exp(m1 - m)
        a2 = jnp.exp(m2 - m)
        out_ref[0, :, hh, :] = (o1 * a1 + o2 * a2) / (l1 * a1 + l2 * a2)

        @pl.when(hh == h - 1)
        def _():
            for i in range(h):
                @pl.when((i % 2) == my_y)
                def _():
                    kc, vc = x_descriptors(i)
                    kc.wait_send()
                    vc.wait_send()
                    rk, rv = y_descriptors(i)
                    rk.wait_send()
                    rv.wait_send()

    vmem = pl.BlockSpec(memory_space=pltpu.MemorySpace.VMEM)
    return pl.pallas_call(
        body,
        grid=(h,),
        out_shape=jax.ShapeDtypeStruct((b, s, h, d), jnp.float32),
        in_specs=[
            vmem,
            vmem,
            vmem,
            pl.BlockSpec(memory_space=pl.ANY),
            pl.BlockSpec(memory_space=pl.ANY),
        ],
        out_specs=vmem,
        scratch_shapes=[
            pltpu.VMEM((h, s, d), jnp.float32),
            pltpu.VMEM((h, s, d), jnp.float32),
            pltpu.SemaphoreType.DMA((h // 2,)),
            pltpu.SemaphoreType.DMA((h // 2,)),
            pltpu.SemaphoreType.DMA((h // 2,)),
            pltpu.SemaphoreType.DMA((h // 2,)),
            pltpu.SemaphoreType.DMA((h // 2,)),
            pltpu.SemaphoreType.DMA((h // 2,)),
            pltpu.SemaphoreType.DMA((h // 2,)),
            pltpu.SemaphoreType.DMA((h // 2,)),
        ],
        compiler_params=pltpu.CompilerParams(
            collective_id=0,
            dimension_semantics=("arbitrary",),
            vmem_limit_bytes=100 * 1024 * 1024,
        ),
    )(Q, K, V, K, V)


# device time: 163958 ns/iter; 2.0553x vs baseline; 2.0553x over previous
import jax
import jax.numpy as jnp
from jax import lax
from jax.experimental import pallas as pl
from jax.experimental.pallas import tpu as pltpu


def kernel(Q, K, V):
    b, s, h, d = Q.shape
    scale = d ** -0.5

    def body(q_ref, k_ref, v_ref, k_any, v_any, out_ref, krem, vrem,
             ksx, krx, ksy, kry, vsx, vrx, vsy, vry):
        step = pl.program_id(0)
        my_x = lax.axis_index("x")
        my_y = lax.axis_index("y")
        my_z = lax.axis_index("z")
        xnbr = (1 - my_x, my_y, my_z)
        ynbr = (my_x, 1 - my_y, my_z)

        on_x = step < (h // 2)
        hh = jnp.where(on_x, 2 * step + my_y,
                       2 * (step - h // 2) + 1 - my_y)

        def x_descriptors(i):
            k_copy = pltpu.make_async_remote_copy(
                src_ref=k_any.at[0, :, i, :],
                dst_ref=krem.at[i],
                send_sem=ksx.at[i // 2],
                recv_sem=krx.at[i // 2],
                device_id=xnbr,
                device_id_type=pl.DeviceIdType.MESH,
            )
            v_copy = pltpu.make_async_remote_copy(
                src_ref=v_any.at[0, :, i, :],
                dst_ref=vrem.at[i],
                send_sem=vsx.at[i // 2],
                recv_sem=vrx.at[i // 2],
                device_id=xnbr,
                device_id_type=pl.DeviceIdType.MESH,
            )
            return k_copy, v_copy

        def y_descriptors(i):
            k_copy = pltpu.make_async_remote_copy(
                src_ref=krem.at[i],
                dst_ref=krem.at[i],
                send_sem=ksy.at[i // 2],
                recv_sem=kry.at[i // 2],
                device_id=ynbr,
                device_id_type=pl.DeviceIdType.MESH,
            )
            v_copy = pltpu.make_async_remote_copy(
                src_ref=vrem.at[i],
                dst_ref=vrem.at[i],
                send_sem=vsy.at[i // 2],
                recv_sem=vry.at[i // 2],
                device_id=ynbr,
                device_id_type=pl.DeviceIdType.MESH,
            )
            return k_copy, v_copy

        @pl.when(step == 0)
        def _():
            barrier = pltpu.get_barrier_semaphore()
            for nbr in (xnbr, ynbr):
                pl.semaphore_signal(barrier, inc=1, device_id=nbr,
                                    device_id_type=pl.DeviceIdType.MESH)
            pl.semaphore_wait(barrier, 2)
            for i in range(h):
                @pl.when((i % 2) == my_y)
                def _():
                    kc, vc = x_descriptors(i)
                    kc.start()
                    vc.start()

        q = q_ref[0, :, hh, :]
        k1 = k_ref[0, :, hh, :]
        v1 = v_ref[0, :, hh, :]
        s1 = lax.dot_general(q, k1, (((1,), (1,)), ((), ())),
                             preferred_element_type=jnp.float32) * scale
        m1 = jnp.max(s1, axis=1, keepdims=True)
        p1 = jnp.exp(s1 - m1)
        l1 = jnp.sum(p1, axis=1, keepdims=True)
        o1 = lax.dot_general(p1, v1, (((1,), (0,)), ((), ())),
                             preferred_element_type=jnp.float32)

        @pl.when(on_x)
        def _():
            kc, vc = x_descriptors(hh)
            kc.wait_recv()
            vc.wait_recv()
            rk, rv = y_descriptors(hh)
            rk.start()
            rv.start()

        @pl.when(jnp.logical_not(on_x))
        def _():
            kc, vc = y_descriptors(hh)
            kc.wait_recv()
            vc.wait_recv()

        k2 = krem[hh]
        v2 = vrem[hh]
        s2 = lax.dot_general(q, k2, (((1,), (1,)), ((), ())),
                             preferred_element_type=jnp.float32) * scale
        m2 = jnp.max(s2, axis=1, keepdims=True)
        p2 = jnp.exp(s2 - m2)
        l2 = jnp.sum(p2, axis=1, keepdims=True)
        o2 = lax.dot_general(p2, v2, (((1,), (0,)), ((), ())),
                             preferred_element_type=jnp.float32)

        m = jnp.maximum(m1, m2)
        a1 = jnp.exp(m1 - m)
        a2 = jnp.exp(m2 - m)
        out_ref[0, :, hh, :] = (o1 * a1 + o2 * a2) / (l1 * a1 + l2 * a2)

        @pl.when(step == h - 1)
        def _():
            for i in range(h):
                @pl.when((i % 2) == my_y)
                def _():
                    kc, vc = x_descriptors(i)
                    kc.wait_send()
                    vc.wait_send()
                    rk, rv = y_descriptors(i)
                    rk.wait_send()
                    rv.wait_send()

    vmem = pl.BlockSpec(memory_space=pltpu.MemorySpace.VMEM)
    return pl.pallas_call(
        body,
        grid=(h,),
        out_shape=jax.ShapeDtypeStruct((b, s, h, d), jnp.float32),
        in_specs=[
            vmem,
            vmem,
            vmem,
            pl.BlockSpec(memory_space=pl.ANY),
            pl.BlockSpec(memory_space=pl.ANY),
        ],
        out_specs=vmem,
        scratch_shapes=[
            pltpu.VMEM((h, s, d), jnp.float32),
            pltpu.VMEM((h, s, d), jnp.float32),
            pltpu.SemaphoreType.DMA((h // 2,)),
            pltpu.SemaphoreType.DMA((h // 2,)),
            pltpu.SemaphoreType.DMA((h // 2,)),
            pltpu.SemaphoreType.DMA((h // 2,)),
            pltpu.SemaphoreType.DMA((h // 2,)),
            pltpu.SemaphoreType.DMA((h // 2,)),
            pltpu.SemaphoreType.DMA((h // 2,)),
            pltpu.SemaphoreType.DMA((h // 2,)),
        ],
        compiler_params=pltpu.CompilerParams(
            collective_id=0,
            dimension_semantics=("arbitrary",),
            vmem_limit_bytes=100 * 1024 * 1024,
        ),
    )(Q, K, V, K, V)


# device time: 90530 ns/iter; 3.7223x vs baseline; 1.8111x over previous
import jax
import jax.numpy as jnp
from jax import lax
from jax.experimental import pallas as pl
from jax.experimental.pallas import tpu as pltpu


def kernel(Q, K, V):
    b, s, h, d = Q.shape
    scale = d ** -0.5

    Kc = jnp.transpose(K, (0, 2, 1, 3)).astype(jnp.bfloat16)
    Vc = jnp.transpose(V, (0, 2, 1, 3)).astype(jnp.bfloat16)

    def body(q_ref, k_ref, v_ref, kc_any, vc_any, out_ref,
             krem, vrem, obuf, ksx, krx, vsx, vrx, osy, ory):
        step = pl.program_id(0)
        my_x = lax.axis_index("x")
        my_y = lax.axis_index("y")
        my_z = lax.axis_index("z")
        xnbr = (1 - my_x, my_y, my_z)
        ynbr = (my_x, 1 - my_y, my_z)

        mine = step < (h // 2)
        hh = jnp.where(mine, 2 * step + my_y,
                       2 * (step - h // 2) + 1 - my_y)

        def x_descriptors(i):
            k_copy = pltpu.make_async_remote_copy(
                src_ref=kc_any.at[0, i],
                dst_ref=krem.at[i],
                send_sem=ksx.at[i // 2],
                recv_sem=krx.at[i // 2],
                device_id=xnbr,
                device_id_type=pl.DeviceIdType.MESH,
            )
            v_copy = pltpu.make_async_remote_copy(
                src_ref=vc_any.at[0, i],
                dst_ref=vrem.at[i],
                send_sem=vsx.at[i // 2],
                recv_sem=vrx.at[i // 2],
                device_id=xnbr,
                device_id_type=pl.DeviceIdType.MESH,
            )
            return k_copy, v_copy

        def o_descriptor(i):
            return pltpu.make_async_remote_copy(
                src_ref=obuf.at[i],
                dst_ref=obuf.at[i],
                send_sem=osy.at[i // 2],
                recv_sem=ory.at[i // 2],
                device_id=ynbr,
                device_id_type=pl.DeviceIdType.MESH,
            )

        @pl.when(step == 0)
        def _():
            barrier = pltpu.get_barrier_semaphore()
            for nbr in (xnbr, ynbr):
                pl.semaphore_signal(barrier, inc=1, device_id=nbr,
                                    device_id_type=pl.DeviceIdType.MESH)
            pl.semaphore_wait(barrier, 2)
            for i in range(h):
                @pl.when((i % 2) == my_y)
                def _():
                    kc, vc = x_descriptors(i)
                    kc.start()
                    vc.start()

        @pl.when(mine)
        def _():
            q = q_ref[0, :, hh, :]
            k1 = k_ref[0, :, hh, :]
            v1 = v_ref[0, :, hh, :]
            s1 = lax.dot_general(q, k1, (((1,), (1,)), ((), ())),
                                 preferred_element_type=jnp.float32) * scale
            m1 = jnp.max(s1, axis=1, keepdims=True)
            p1 = jnp.exp(s1 - m1)
            l1 = jnp.sum(p1, axis=1, keepdims=True)
            o1 = lax.dot_general(p1, v1, (((1,), (0,)), ((), ())),
                                 preferred_element_type=jnp.float32)

            kc, vc = x_descriptors(hh)
            kc.wait_recv()
            vc.wait_recv()
            k2 = krem[hh].astype(jnp.float32)
            v2 = vrem[hh].astype(jnp.float32)
            s2 = lax.dot_general(q, k2, (((1,), (1,)), ((), ())),
                                 preferred_element_type=jnp.float32) * scale
            m2 = jnp.max(s2, axis=1, keepdims=True)
            p2 = jnp.exp(s2 - m2)
            l2 = jnp.sum(p2, axis=1, keepdims=True)
            o2 = lax.dot_general(p2, v2, (((1,), (0,)), ((), ())),
                                 preferred_element_type=jnp.float32)

            m = jnp.maximum(m1, m2)
            a1 = jnp.exp(m1 - m)
            a2 = jnp.exp(m2 - m)
            o = (o1 * a1 + o2 * a2) / (l1 * a1 + l2 * a2)
            out_ref[0, :, hh, :] = o
            obuf[hh] = o.astype(jnp.bfloat16)
            o_descriptor(hh).start()

        @pl.when(jnp.logical_not(mine))
        def _():
            oc = o_descriptor(hh)
            oc.wait_recv()
            out_ref[0, :, hh, :] = obuf[hh].astype(jnp.float32)

        @pl.when(step == h - 1)
        def _():
            for i in range(h):
                @pl.when((i % 2) == my_y)
                def _():
                    kc, vc = x_descriptors(i)
                    kc.wait_send()
                    vc.wait_send()
                    o_descriptor(i).wait_send()

    vmem = pl.BlockSpec(memory_space=pltpu.MemorySpace.VMEM)
    return pl.pallas_call(
        body,
        grid=(h,),
        out_shape=jax.ShapeDtypeStruct((b, s, h, d), jnp.float32),
        in_specs=[
            vmem,
            vmem,
            vmem,
            pl.BlockSpec(memory_space=pl.ANY),
            pl.BlockSpec(memory_space=pl.ANY),
        ],
        out_specs=vmem,
        scratch_shapes=[
            pltpu.VMEM((h, s, d), jnp.bfloat16),
            pltpu.VMEM((h, s, d), jnp.bfloat16),
            pltpu.VMEM((h, s, d), jnp.bfloat16),
            pltpu.SemaphoreType.DMA((h // 2,)),
            pltpu.SemaphoreType.DMA((h // 2,)),
            pltpu.SemaphoreType.DMA((h // 2,)),
            pltpu.SemaphoreType.DMA((h // 2,)),
            pltpu.SemaphoreType.DMA((h // 2,)),
            pltpu.SemaphoreType.DMA((h // 2,)),
        ],
        compiler_params=pltpu.CompilerParams(
            collective_id=0,
            dimension_semantics=("arbitrary",),
            vmem_limit_bytes=100 * 1024 * 1024,
        ),
    )(Q, K, V, Kc, Vc)


# device time: 84010 ns/iter; 4.0111x vs baseline; 1.0776x over previous
import jax
import jax.numpy as jnp
from jax import lax
from jax.experimental import pallas as pl
from jax.experimental.pallas import tpu as pltpu

N_YZ = 8


def kernel(Q, K, V):
    b, s, h, d = Q.shape
    scale = d ** -0.5
    hpd = h // N_YZ

    def body(q_ref, k_ref, v_ref, out_ref,
             kstage, vstage, krem, vrem, obuf,
             ksx, krx, vsx, vrx, osy, ory):
        step = pl.program_id(0)
        my_x = lax.axis_index("x")
        my_y = lax.axis_index("y")
        my_z = lax.axis_index("z")
        xnbr = (1 - my_x, my_y, my_z)
        r = my_y * 4 + my_z

        def peer(p):
            return (my_x, p // 4, p % 4)

        mine = step < hpd
        fj = step - hpd
        hh = jnp.where(mine, hpd * r + step, fj + hpd * (fj >= hpd * r))

        def kv_descriptors(j):
            k_copy = pltpu.make_async_remote_copy(
                src_ref=kstage.at[j],
                dst_ref=krem.at[j],
                send_sem=ksx.at[j],
                recv_sem=krx.at[j],
                device_id=xnbr,
                device_id_type=pl.DeviceIdType.MESH,
            )
            v_copy = pltpu.make_async_remote_copy(
                src_ref=vstage.at[j],
                dst_ref=vrem.at[j],
                send_sem=vsx.at[j],
                recv_sem=vrx.at[j],
                device_id=xnbr,
                device_id_type=pl.DeviceIdType.MESH,
            )
            return k_copy, v_copy

        def o_send(j, p):
            return pltpu.make_async_remote_copy(
                src_ref=obuf.at[hpd * r + j],
                dst_ref=obuf.at[hpd * r + j],
                send_sem=osy.at[p, j],
                recv_sem=ory.at[r, j],
                device_id=peer(p),
                device_id_type=pl.DeviceIdType.MESH,
            )

        def o_recv(g):
            return pltpu.make_async_remote_copy(
                src_ref=obuf.at[g],
                dst_ref=obuf.at[g],
                send_sem=osy.at[g // hpd, g % hpd],
                recv_sem=ory.at[g // hpd, g % hpd],
                device_id=peer(g // hpd),
                device_id_type=pl.DeviceIdType.MESH,
            )

        @pl.when(step == 0)
        def _():
            barrier = pltpu.get_barrier_semaphore()
            pl.semaphore_signal(barrier, inc=1, device_id=xnbr,
                                device_id_type=pl.DeviceIdType.MESH)
            for o in range(1, N_YZ):
                pl.semaphore_signal(barrier, inc=1,
                                    device_id=peer((r + o) % N_YZ),
                                    device_id_type=pl.DeviceIdType.MESH)
            pl.semaphore_wait(barrier, N_YZ)
            for j in range(hpd):
                g = hpd * r + j
                kstage[j] = k_ref[0, :, g, :].astype(jnp.bfloat16)
                vstage[j] = v_ref[0, :, g, :].astype(jnp.bfloat16)
                kc, vc = kv_descriptors(j)
                kc.start()
                vc.start()

        @pl.when(mine)
        def _():
            q = q_ref[0, :, hh, :]
            k1 = k_ref[0, :, hh, :]
            v1 = v_ref[0, :, hh, :]
            s1 = lax.dot_general(q, k1, (((1,), (1,)), ((), ())),
                                 preferred_element_type=jnp.float32) * scale
            m1 = jnp.max(s1, axis=1, keepdims=True)
            p1 = jnp.exp(s1 - m1)
            l1 = jnp.sum(p1, axis=1, keepdims=True)
            o1 = lax.dot_general(p1, v1, (((1,), (0,)), ((), ())),
                                 preferred_element_type=jnp.float32)

            kc, vc = kv_descriptors(step)
            kc.wait_recv()
            vc.wait_recv()
            k2 = krem[step].astype(jnp.float32)
            v2 = vrem[step].astype(jnp.float32)
            s2 = lax.dot_general(q, k2, (((1,), (1,)), ((), ())),
                                 preferred_element_type=jnp.float32) * scale
            m2 = jnp.max(s2, axis=1, keepdims=True)
            p2 = jnp.exp(s2 - m2)
            l2 = jnp.sum(p2, axis=1, keepdims=True)
            o2 = lax.dot_general(p2, v2, (((1,), (0,)), ((), ())),
                                 preferred_element_type=jnp.float32)

            m = jnp.maximum(m1, m2)
            a1 = jnp.exp(m1 - m)
            a2 = jnp.exp(m2 - m)
            o = (o1 * a1 + o2 * a2) / (l1 * a1 + l2 * a2)
            out_ref[0, :, hh, :] = o
            obuf[hh] = o.astype(jnp.bfloat16)
            for off in range(1, N_YZ):
                o_send(step, (r + off) % N_YZ).start()

        @pl.when(jnp.logical_not(mine))
        def _():
            oc = o_recv(hh)
            oc.wait_recv()
            out_ref[0, :, hh, :] = obuf[hh].astype(jnp.float32)

        @pl.when(step == h - 1)
        def _():
            for j in range(hpd):
                kc, vc = kv_descriptors(j)
                kc.wait_send()
                vc.wait_send()
                for off in range(1, N_YZ):
                    o_send(j, (r + off) % N_YZ).wait_send()

    vmem = pl.BlockSpec(memory_space=pltpu.MemorySpace.VMEM)
    return pl.pallas_call(
        body,
        grid=(h,),
        out_shape=jax.ShapeDtypeStruct((b, s, h, d), jnp.float32),
        in_specs=[vmem, vmem, vmem],
        out_specs=vmem,
        scratch_shapes=[
            pltpu.VMEM((hpd, s, d), jnp.bfloat16),
            pltpu.VMEM((hpd, s, d), jnp.bfloat16),
            pltpu.VMEM((hpd, s, d), jnp.bfloat16),
            pltpu.VMEM((hpd, s, d), jnp.bfloat16),
            pltpu.VMEM((h, s, d), jnp.bfloat16),
            pltpu.SemaphoreType.DMA((hpd,)),
            pltpu.SemaphoreType.DMA((hpd,)),
            pltpu.SemaphoreType.DMA((hpd,)),
            pltpu.SemaphoreType.DMA((hpd,)),
            pltpu.SemaphoreType.DMA((N_YZ, hpd)),
            pltpu.SemaphoreType.DMA((N_YZ, hpd)),
        ],
        compiler_params=pltpu.CompilerParams(
            collective_id=0,
            dimension_semantics=("arbitrary",),
            vmem_limit_bytes=100 * 1024 * 1024,
        ),
    )(Q, K, V)


# device time: 81235 ns/iter; 4.1482x vs baseline; 1.0342x over previous
import jax
import jax.numpy as jnp
from jax import lax
from jax.experimental import pallas as pl
from jax.experimental.pallas import tpu as pltpu

N_YZ = 8


def kernel(Q, K, V):
    b, s, h, d = Q.shape
    scale = d ** -0.5
    hpd = h // N_YZ

    def body(q_ref, k_ref, v_ref, out_ref,
             kstage, vstage, krem, vrem, obuf,
             ksx, krx, vsx, vrx, osy, ory):
        my_x = lax.axis_index("x")
        my_y = lax.axis_index("y")
        my_z = lax.axis_index("z")
        xnbr = (1 - my_x, my_y, my_z)
        r = my_y * 4 + my_z

        def peer(p):
            return (my_x, p // 4, p % 4)

        def kv_descriptors(j):
            k_copy = pltpu.make_async_remote_copy(
                src_ref=kstage.at[j],
                dst_ref=krem.at[j],
                send_sem=ksx.at[j],
                recv_sem=krx.at[j],
                device_id=xnbr,
                device_id_type=pl.DeviceIdType.MESH,
            )
            v_copy = pltpu.make_async_remote_copy(
                src_ref=vstage.at[j],
                dst_ref=vrem.at[j],
                send_sem=vsx.at[j],
                recv_sem=vrx.at[j],
                device_id=xnbr,
                device_id_type=pl.DeviceIdType.MESH,
            )
            return k_copy, v_copy

        def o_send(j, p):
            return pltpu.make_async_remote_copy(
                src_ref=obuf.at[hpd * r + j],
                dst_ref=obuf.at[hpd * r + j],
                send_sem=osy.at[p, j],
                recv_sem=ory.at[r, j],
                device_id=peer(p),
                device_id_type=pl.DeviceIdType.MESH,
            )

        def o_recv(g):
            return pltpu.make_async_remote_copy(
                src_ref=obuf.at[g],
                dst_ref=obuf.at[g],
                send_sem=osy.at[g // hpd, g % hpd],
                recv_sem=ory.at[g // hpd, g % hpd],
                device_id=peer(g // hpd),
                device_id_type=pl.DeviceIdType.MESH,
            )

        barrier = pltpu.get_barrier_semaphore()
        pl.semaphore_signal(barrier, inc=1, device_id=xnbr,
                            device_id_type=pl.DeviceIdType.MESH)
        for o in range(1, N_YZ):
            pl.semaphore_signal(barrier, inc=1,
                                device_id=peer((r + o) % N_YZ),
                                device_id_type=pl.DeviceIdType.MESH)
        pl.semaphore_wait(barrier, N_YZ)

        for j in range(hpd):
            g = hpd * r + j
            kstage[j] = k_ref[0, :, g, :].astype(jnp.bfloat16)
            vstage[j] = v_ref[0, :, g, :].astype(jnp.bfloat16)
            kc, vc = kv_descriptors(j)
            kc.start()
            vc.start()

        for j in range(hpd):
            g = hpd * r + j
            q = q_ref[0, :, g, :]
            k1 = k_ref[0, :, g, :]
            v1 = v_ref[0, :, g, :]
            s1 = lax.dot_general(q, k1, (((1,), (1,)), ((), ())),
                                 preferred_element_type=jnp.float32) * scale
            m1 = jnp.max(s1, axis=1, keepdims=True)
            p1 = jnp.exp(s1 - m1)
            l1 = jnp.sum(p1, axis=1, keepdims=True)
            o1 = lax.dot_general(p1, v1, (((1,), (0,)), ((), ())),
                                 preferred_element_type=jnp.float32)

            kc, vc = kv_descriptors(j)
            kc.wait_recv()
            vc.wait_recv()
            k2 = krem[j].astype(jnp.float32)
            v2 = vrem[j].astype(jnp.float32)
            s2 = lax.dot_general(q, k2, (((1,), (1,)), ((), ())),
                                 preferred_element_type=jnp.float32) * scale
            m2 = jnp.max(s2, axis=1, keepdims=True)
            p2 = jnp.exp(s2 - m2)
            l2 = jnp.sum(p2, axis=1, keepdims=True)
            o2 = lax.dot_general(p2, v2, (((1,), (0,)), ((), ())),
                                 preferred_element_type=jnp.float32)

            m = jnp.maximum(m1, m2)
            a1 = jnp.exp(m1 - m)
            a2 = jnp.exp(m2 - m)
            o = (o1 * a1 + o2 * a2) / (l1 * a1 + l2 * a2)
            obuf[g] = o.astype(jnp.bfloat16)
            for off in range(1, N_YZ):
                o_send(j, (r + off) % N_YZ).start()

        for fj in range(h - hpd):
            g = fj + hpd * (fj >= hpd * r)
            o_recv(g).wait_recv()

        out_ref[0, :, :, :] = jnp.transpose(
            obuf[...], (1, 0, 2)).astype(jnp.float32)

        for j in range(hpd):
            kc, vc = kv_descriptors(j)
            kc.wait_send()
            vc.wait_send()
            for off in range(1, N_YZ):
                o_send(j, (r + off) % N_YZ).wait_send()

    vmem = pl.BlockSpec(memory_space=pltpu.MemorySpace.VMEM)
    return pl.pallas_call(
        body,
        out_shape=jax.ShapeDtypeStruct((b, s, h, d), jnp.float32),
        in_specs=[vmem, vmem, vmem],
        out_specs=vmem,
        scratch_shapes=[
            pltpu.VMEM((hpd, s, d), jnp.bfloat16),
            pltpu.VMEM((hpd, s, d), jnp.bfloat16),
            pltpu.VMEM((hpd, s, d), jnp.bfloat16),
            pltpu.VMEM((hpd, s, d), jnp.bfloat16),
            pltpu.VMEM((h, s, d), jnp.bfloat16),
            pltpu.SemaphoreType.DMA((hpd,)),
            pltpu.SemaphoreType.DMA((hpd,)),
            pltpu.SemaphoreType.DMA((hpd,)),
            pltpu.SemaphoreType.DMA((hpd,)),
            pltpu.SemaphoreType.DMA((N_YZ, hpd)),
            pltpu.SemaphoreType.DMA((N_YZ, hpd)),
        ],
        compiler_params=pltpu.CompilerParams(
            collective_id=0,
            vmem_limit_bytes=100 * 1024 * 1024,
        ),
    )(Q, K, V)


# device time: 62596 ns/iter; 5.3833x vs baseline; 1.2978x over previous
import jax
import jax.numpy as jnp
from jax import lax
from jax.experimental import pallas as pl
from jax.experimental.pallas import tpu as pltpu

N_GRP = 4


def kernel(Q, K, V):
    b, s, h, d = Q.shape
    scale = d ** -0.5
    hpd = h // N_GRP

    def body(q_ref, k_ref, v_ref, out_ref,
             kstage, vstage, krem, vrem, obuf,
             ksx, krx, vsx, vrx, osy, ory):
        my_x = lax.axis_index("x")
        my_y = lax.axis_index("y")
        my_z = lax.axis_index("z")
        xnbr = (1 - my_x, my_y, my_z)
        zbase = (my_z // 2) * 2
        r = my_y * 2 + (my_z % 2)

        def peer(p):
            return (my_x, p // 2, zbase + p % 2)

        def kv_descriptors(j):
            k_copy = pltpu.make_async_remote_copy(
                src_ref=kstage.at[j],
                dst_ref=krem.at[j],
                send_sem=ksx.at[j],
                recv_sem=krx.at[j],
                device_id=xnbr,
                device_id_type=pl.DeviceIdType.MESH,
            )
            v_copy = pltpu.make_async_remote_copy(
                src_ref=vstage.at[j],
                dst_ref=vrem.at[j],
                send_sem=vsx.at[j],
                recv_sem=vrx.at[j],
                device_id=xnbr,
                device_id_type=pl.DeviceIdType.MESH,
            )
            return k_copy, v_copy

        def o_send(j, p):
            return pltpu.make_async_remote_copy(
                src_ref=obuf.at[hpd * r + j],
                dst_ref=obuf.at[hpd * r + j],
                send_sem=osy.at[p, j],
                recv_sem=ory.at[r, j],
                device_id=peer(p),
                device_id_type=pl.DeviceIdType.MESH,
            )

        def o_recv(g):
            return pltpu.make_async_remote_copy(
                src_ref=obuf.at[g],
                dst_ref=obuf.at[g],
                send_sem=osy.at[g // hpd, g % hpd],
                recv_sem=ory.at[g // hpd, g % hpd],
                device_id=peer(g // hpd),
                device_id_type=pl.DeviceIdType.MESH,
            )

        barrier = pltpu.get_barrier_semaphore()
        pl.semaphore_signal(barrier, inc=1, device_id=xnbr,
                            device_id_type=pl.DeviceIdType.MESH)
        for o in range(1, N_GRP):
            pl.semaphore_signal(barrier, inc=1,
                                device_id=peer((r + o) % N_GRP),
                                device_id_type=pl.DeviceIdType.MESH)
        pl.semaphore_wait(barrier, N_GRP)

        for j in range(hpd):
            g = hpd * r + j
            kstage[j] = k_ref[0, :, g, :].astype(jnp.bfloat16)
            vstage[j] = v_ref[0, :, g, :].astype(jnp.bfloat16)
            kc, vc = kv_descriptors(j)
            kc.start()
            vc.start()

        for j in range(hpd):
            g = hpd * r + j
            q = q_ref[0, :, g, :]
            k1 = k_ref[0, :, g, :]
            v1 = v_ref[0, :, g, :]
            s1 = lax.dot_general(q, k1, (((1,), (1,)), ((), ())),
                                 preferred_element_type=jnp.float32) * scale
            m1 = jnp.max(s1, axis=1, keepdims=True)
            p1 = jnp.exp(s1 - m1)
            l1 = jnp.sum(p1, axis=1, keepdims=True)
            o1 = lax.dot_general(p1, v1, (((1,), (0,)), ((), ())),
                                 preferred_element_type=jnp.float32)

            kc, vc = kv_descriptors(j)
            kc.wait_recv()
            vc.wait_recv()
            k2 = krem[j].astype(jnp.float32)
            v2 = vrem[j].astype(jnp.float32)
            s2 = lax.dot_general(q, k2, (((1,), (1,)), ((), ())),
                                 preferred_element_type=jnp.float32) * scale
            m2 = jnp.max(s2, axis=1, keepdims=True)
            p2 = jnp.exp(s2 - m2)
            l2 = jnp.sum(p2, axis=1, keepdims=True)
            o2 = lax.dot_general(p2, v2, (((1,), (0,)), ((), ())),
                                 preferred_element_type=jnp.float32)

            m = jnp.maximum(m1, m2)
            a1 = jnp.exp(m1 - m)
            a2 = jnp.exp(m2 - m)
            o = (o1 * a1 + o2 * a2) / (l1 * a1 + l2 * a2)
            obuf[g] = o.astype(jnp.bfloat16)
            for off in range(1, N_GRP):
                o_send(j, (r + off) % N_GRP).start()

        for fj in range(h - hpd):
            g = fj + hpd * (fj >= hpd * r)
            o_recv(g).wait_recv()

        out_ref[0, :, :, :] = jnp.transpose(
            obuf[...], (1, 0, 2)).astype(jnp.float32)

        for j in range(hpd):
            kc, vc = kv_descriptors(j)
            kc.wait_send()
            vc.wait_send()
            for off in range(1, N_GRP):
                o_send(j, (r + off) % N_GRP).wait_send()

    vmem = pl.BlockSpec(memory_space=pltpu.MemorySpace.VMEM)
    return pl.pallas_call(
        body,
        out_shape=jax.ShapeDtypeStruct((b, s, h, d), jnp.float32),
        in_specs=[vmem, vmem, vmem],
        out_specs=vmem,
        scratch_shapes=[
            pltpu.VMEM((hpd, s, d), jnp.bfloat16),
            pltpu.VMEM((hpd, s, d), jnp.bfloat16),
            pltpu.VMEM((hpd, s, d), jnp.bfloat16),
            pltpu.VMEM((hpd, s, d), jnp.bfloat16),
            pltpu.VMEM((h, s, d), jnp.bfloat16),
            pltpu.SemaphoreType.DMA((hpd,)),
            pltpu.SemaphoreType.DMA((hpd,)),
            pltpu.SemaphoreType.DMA((hpd,)),
            pltpu.SemaphoreType.DMA((hpd,)),
            pltpu.SemaphoreType.DMA((N_GRP, hpd)),
            pltpu.SemaphoreType.DMA((N_GRP, hpd)),
        ],
        compiler_params=pltpu.CompilerParams(
            collective_id=0,
            vmem_limit_bytes=100 * 1024 * 1024,
        ),
    )(Q, K, V)
